# baseline (speedup 1.0000x reference)
import functools

import jax
import jax.numpy as jnp
from jax import lax
from jax.experimental import pallas as pl
from jax.experimental.pallas import tpu as pltpu
from jax.experimental.pallas import tpu_sc as plsc

N_NODES = 10000
N_EDGES = 320000
D = 128

NC = 2
NS = 16
NW = NC * NS
LANES = 16

B = 80
NB = 126
E_PAD = NW * NB * B
IDX_RING = 6
GROUPS = NB // IDX_RING
ROW_CHUNK = 80
NUM_ROW_CHUNKS = N_NODES // ROW_CHUNK


def _mm_combine_body(p_ref, w_ref, b_ref, o_ref):
    s = p_ref[0] + p_ref[1]
    o_ref[...] = jnp.dot(s, w_ref[...],
                         preferred_element_type=jnp.float32) + b_ref[...]


def _tc_mm_combine(partials, w, bias2d):
    return pl.pallas_call(
        _mm_combine_body,
        out_shape=jax.ShapeDtypeStruct((N_NODES, D), jnp.float32),
    )(partials, w, bias2d)


def _spmm_sc(x, src, dst, vals):
    mesh = plsc.VectorSubcoreMesh(core_axis_name="c", subcore_axis_name="s")

    @functools.partial(
        pl.kernel,
        out_type=jax.ShapeDtypeStruct((NC, N_NODES, D), jnp.float32),
        mesh=mesh,
        scratch_types=[
            pltpu.VMEM((IDX_RING, B), jnp.int32),
            pltpu.VMEM((IDX_RING, B), jnp.int32),
            pltpu.VMEM((IDX_RING, B), jnp.float32),
            pltpu.VMEM((B, D), jnp.float32),
            pltpu.VMEM((B, D), jnp.float32),
            pltpu.VMEM((B, D), jnp.float32),
            pltpu.VMEM_SHARED((N_NODES, D), jnp.float32),
            pltpu.SemaphoreType.DMA((IDX_RING,)),
            pltpu.SemaphoreType.DMA((3,)),
            pltpu.SemaphoreType.DMA((3,)),
        ],
    )
    def k(x_hbm, src_hbm, dst_hbm, vals_hbm, out_hbm,
          src_r, dst_r, val_r, rows0, rows1, rows2, acc_sh,
          sem_i, sem_g, sem_s):
        cid = lax.axis_index("c")
        sid = lax.axis_index("s")
        wid = cid * NS + sid
        ebase = wid * (NB * B)

        rows = (rows0, rows1, rows2)

        def start_idx(kb, q):
            off = ebase + kb * B
            pltpu.async_copy(src_hbm.at[pl.ds(off, B)], src_r.at[q],
                             sem_i.at[q])
            pltpu.async_copy(dst_hbm.at[pl.ds(off, B)], dst_r.at[q],
                             sem_i.at[q])
            pltpu.async_copy(vals_hbm.at[pl.ds(off, B)], val_r.at[q],
                             sem_i.at[q])

        def wait_idx(q):
            pltpu.make_async_copy(src_hbm.at[pl.ds(0, B)], src_r.at[q],
                                  sem_i.at[q]).wait()
            pltpu.make_async_copy(dst_hbm.at[pl.ds(0, B)], dst_r.at[q],
                                  sem_i.at[q]).wait()
            pltpu.make_async_copy(vals_hbm.at[pl.ds(0, B)], val_r.at[q],
                                  sem_i.at[q]).wait()

        def start_gather(r, q):
            pltpu.async_copy(x_hbm.at[src_r.at[q]], rows[r], sem_g.at[r])

        def wait_gather(r):
            pltpu.make_async_copy(x_hbm.at[pl.ds(0, B)], rows[r],
                                  sem_g.at[r]).wait()

        def start_scatter(r, q):
            pltpu.async_copy(rows[r], acc_sh.at[dst_r.at[q]], sem_s.at[r],
                             add=True)

        def wait_scatter(r):
            pltpu.make_async_copy(rows[r], acc_sh.at[pl.ds(0, B)],
                                  sem_s.at[r]).wait()

        def scale(r, q):
            buf = rows[r]

            @pl.loop(0, B, step=LANES)
            def _(g):
                vvec = val_r[q, pl.ds(g, LANES)]
                for i in range(LANES):
                    v = vvec[i]
                    for c in range(D // LANES):
                        sl = pl.ds(c * LANES, LANES)
                        buf[g + i, sl] = buf[g + i, sl] * v

        zvec = jnp.zeros((LANES,), jnp.float32)

        @pl.loop(0, ROW_CHUNK)
        def _(i):
            for c in range(D // LANES):
                rows0[i, pl.ds(c * LANES, LANES)] = zvec

        @pl.loop(sid, NUM_ROW_CHUNKS, step=NS)
        def _(r):
            pltpu.sync_copy(rows0, acc_sh.at[pl.ds(r * ROW_CHUNK,
                                                   ROW_CHUNK)])

        plsc.subcore_barrier()

        for q in range(5):
            start_idx(q, q)
        wait_idx(0)
        start_gather(0, 0)
        wait_idx(1)
        start_gather(1, 1)

        @pl.loop(0, GROUPS)
        def _(grp):
            kbase = IDX_RING * grp
            for t in range(IDX_RING):
                kb = kbase + t
                r = t % 3
                q = t
                wait_gather(r)
                scale(r, q)
                start_scatter(r, q)

                if t == 0:
                    @pl.when(kbase >= 1)
                    def _():
                        wait_scatter(2)
                else:
                    wait_scatter((t + 2) % 3)

                @pl.when(kb + 5 < NB)
                def _():
                    start_idx(kb + 5, (t + 5) % IDX_RING)

                @pl.when(kb + 2 < NB)
                def _():
                    wait_idx((t + 2) % IDX_RING)
                    start_gather((t + 2) % 3, (t + 2) % IDX_RING)

        wait_scatter((NB - 1) % 3)
        plsc.subcore_barrier()

        @pl.loop(sid, NUM_ROW_CHUNKS, step=NS)
        def _(r):
            dbase = r * ROW_CHUNK
            pltpu.sync_copy(acc_sh.at[pl.ds(dbase, ROW_CHUNK)],
                            out_hbm.at[cid, pl.ds(dbase, ROW_CHUNK)])

    return k(x, src, dst, vals)


def kernel(sparse_poly_edge_index, sparse_poly_values, input_feature,
           weight, bias):
    dst = sparse_poly_edge_index[0].astype(jnp.int32)
    src = sparse_poly_edge_index[1].astype(jnp.int32)
    pad = E_PAD - N_EDGES
    zi = jnp.zeros((pad,), jnp.int32)
    src_p = jnp.concatenate([src, zi])
    dst_p = jnp.concatenate([dst, zi])
    vals_p = jnp.concatenate([sparse_poly_values,
                              jnp.zeros((pad,), jnp.float32)])
    partials = _spmm_sc(input_feature, src_p, dst_p, vals_p)
    return _tc_mm_combine(partials, weight, bias.reshape(1, D))
